# baseline (speedup 1.0000x reference)
import jax
import jax.numpy as jnp
from jax.experimental import pallas as pl
from jax.experimental.pallas import tpu as pltpu

B, N, C = 8, 20000, 80
CSPLIT = 2
BC = C // CSPLIT


def _fused_kernel(logits_ref, cent_ref, regress_ref, pts_ref,
                  logits_out_ref, boxes_out_ref):
    l = logits_ref[...]
    c = cent_ref[...]
    logits_out_ref[...] = jnp.sqrt(l * c)

    @pl.when(pl.program_id(1) == 0)
    def _():
        r = regress_ref[...]
        px = pts_ref[0:1, :][None]
        py = pts_ref[1:2, :][None]
        row = jax.lax.broadcasted_iota(jnp.int32, r.shape, 1)
        sign = jnp.where(row >= 2, 1.0, -1.0).astype(jnp.float32)
        pts4 = jnp.where(row % 2 == 0, px, py)
        boxes_out_ref[...] = jnp.clip(pts4 + sign * r, 0.0, 1.0)


def kernel(logits, regress, points, centerness):
    lt = jnp.transpose(logits, (0, 2, 1))
    rt = jnp.transpose(regress, (0, 2, 1))
    pt = jnp.transpose(points, (1, 0))
    ct = jnp.transpose(centerness, (0, 2, 1))

    out = pl.pallas_call(
        _fused_kernel,
        grid=(B, CSPLIT),
        in_specs=[
            pl.BlockSpec((1, BC, N), lambda b, j: (b, j, 0)),
            pl.BlockSpec((1, 1, N), lambda b, j: (b, 0, 0)),
            pl.BlockSpec((1, 4, N), lambda b, j: (b, 0, 0)),
            pl.BlockSpec((2, N), lambda b, j: (0, 0)),
        ],
        out_specs=[
            pl.BlockSpec((1, BC, N), lambda b, j: (b, j, 0)),
            pl.BlockSpec((1, 4, N), lambda b, j: (b, 0, 0)),
        ],
        out_shape=[
            jax.ShapeDtypeStruct((B, C, N), jnp.float32),
            jax.ShapeDtypeStruct((B, 4, N), jnp.float32),
        ],
        compiler_params=pltpu.CompilerParams(
            dimension_semantics=("parallel", "arbitrary"),
        ),
    )(lt, ct, rt, pt)
    return (jnp.transpose(out[0], (0, 2, 1)), jnp.transpose(out[1], (0, 2, 1)))


# speedup vs baseline: 1.0588x; 1.0588x over previous
import jax
import jax.numpy as jnp
from jax.experimental import pallas as pl
from jax.experimental.pallas import tpu as pltpu

B, N, C = 8, 20000, 80
CSPLIT = 1
BC = C // CSPLIT


def _fused_kernel(logits_ref, cent_ref, regress_ref, pts_ref,
                  logits_out_ref, boxes_out_ref):
    l = logits_ref[...]
    c = cent_ref[...]
    logits_out_ref[...] = jnp.sqrt(l * c)

    @pl.when(pl.program_id(1) == 0)
    def _():
        r = regress_ref[...]
        px = pts_ref[0:1, :][None]
        py = pts_ref[1:2, :][None]
        row = jax.lax.broadcasted_iota(jnp.int32, r.shape, 1)
        sign = jnp.where(row >= 2, 1.0, -1.0).astype(jnp.float32)
        pts4 = jnp.where(row % 2 == 0, px, py)
        boxes_out_ref[...] = jnp.clip(pts4 + sign * r, 0.0, 1.0)


def kernel(logits, regress, points, centerness):
    lt = jnp.transpose(logits, (0, 2, 1))
    rt = jnp.transpose(regress, (0, 2, 1))
    pt = jnp.transpose(points, (1, 0))
    ct = jnp.transpose(centerness, (0, 2, 1))

    out = pl.pallas_call(
        _fused_kernel,
        grid=(B, CSPLIT),
        in_specs=[
            pl.BlockSpec((1, BC, N), lambda b, j: (b, j, 0)),
            pl.BlockSpec((1, 1, N), lambda b, j: (b, 0, 0)),
            pl.BlockSpec((1, 4, N), lambda b, j: (b, 0, 0)),
            pl.BlockSpec((2, N), lambda b, j: (0, 0)),
        ],
        out_specs=[
            pl.BlockSpec((1, BC, N), lambda b, j: (b, j, 0)),
            pl.BlockSpec((1, 4, N), lambda b, j: (b, 0, 0)),
        ],
        out_shape=[
            jax.ShapeDtypeStruct((B, C, N), jnp.float32),
            jax.ShapeDtypeStruct((B, 4, N), jnp.float32),
        ],
        compiler_params=pltpu.CompilerParams(
            dimension_semantics=("parallel", "arbitrary"),
        ),
    )(lt, ct, rt, pt)
    return (jnp.transpose(out[0], (0, 2, 1)), jnp.transpose(out[1], (0, 2, 1)))
